# baseline (speedup 1.0000x reference)
import jax
import jax.numpy as jnp
from jax import lax
from jax.experimental import pallas as pl
from jax.experimental.pallas import tpu as pltpu
from jax.experimental.pallas import tpu_sc as plsc

VOCAB = 1000000
EMBED = 64
BATCH = 16384
CTX = 20

NC = 2
NS = 16
NW = NC * NS
BPW = BATCH // NW
CB = 32
NCHUNK = BPW // CB
ROWS = CB * CTX
GCH = 128
NG = ROWS // GCH
NE = EMBED // 16
BG = 4
NQ = CB // BG


def _body(tgt_hbm, ctxflat_hbm, ttab_hbm, ctab_hbm, out_hbm,
          tgt_idx, ctx_idx, trow, crow, accs, zbuf, sem):
    wid = lax.axis_index("s") * NC + lax.axis_index("c")
    wbase = wid * BPW
    lanes = lax.iota(jnp.int32, 16)

    def chunk(g, carry):
        base = wbase + g * CB
        pltpu.sync_copy(tgt_hbm.at[pl.ds(base, CB)], tgt_idx)
        pltpu.sync_copy(ctxflat_hbm.at[pl.ds(base * CTX, ROWS)], ctx_idx)
        cps = [pltpu.async_copy(ttab_hbm.at[tgt_idx], trow, sem)]
        for j in range(NG):
            cps.append(pltpu.async_copy(
                ctab_hbm.at[ctx_idx.at[pl.ds(j * GCH, GCH)]],
                crow.at[pl.ds(j * GCH, GCH)], sem))
        for cp in cps:
            cp.wait()

        def bt(q, c2):
            b0 = q * BG
            t = [[trow[b0 + j, pl.ds(k * 16, 16)] for k in range(NE)]
                 for j in range(BG)]
            zbase = (g * CB + b0) * CTX
            for h in range(BG * CTX // 16):
                for i in range(16):
                    p = h * 16 + i
                    tj = t[p // CTX]
                    acc = crow[b0 * CTX + p, pl.ds(0, 16)] * tj[0]
                    for k in range(1, NE):
                        acc = acc + crow[b0 * CTX + p, pl.ds(k * 16, 16)] * tj[k]
                    accs[i, :] = acc
                zvec = plsc.load_gather(
                    accs, [lanes, jnp.zeros((16,), jnp.int32)])
                for l in range(1, 16):
                    zvec = zvec + plsc.load_gather(
                        accs, [lanes, jnp.full((16,), l, jnp.int32)])
                zbuf[pl.ds(zbase + h * 16, 16)] = zvec
            return c2

        return lax.fori_loop(0, NQ, bt, carry)

    lax.fori_loop(0, NCHUNK, chunk, 0)
    pltpu.sync_copy(zbuf, out_hbm.at[pl.ds(wbase * CTX, BPW * CTX)])


def kernel(targets, contexts, target_table, context_table):
    mesh = plsc.VectorSubcoreMesh(core_axis_name="c", subcore_axis_name="s")
    k = pl.kernel(
        _body,
        out_type=jax.ShapeDtypeStruct((BATCH * CTX,), jnp.float32),
        mesh=mesh,
        compiler_params=pltpu.CompilerParams(
            needs_layout_passes=False, use_tc_tiling_on_sc=False),
        scratch_types=[
            pltpu.VMEM((CB,), jnp.int32),
            pltpu.VMEM((ROWS,), jnp.int32),
            pltpu.VMEM((CB, EMBED), jnp.float32),
            pltpu.VMEM((ROWS, EMBED), jnp.float32),
            pltpu.VMEM((16, 16), jnp.float32),
            pltpu.VMEM((BPW * CTX,), jnp.float32),
            pltpu.SemaphoreType.DMA,
        ],
    )
    z = k(targets.astype(jnp.int32),
          contexts.reshape(-1).astype(jnp.int32),
          target_table, context_table)
    return z.reshape(BATCH, CTX)
